# baseline (speedup 1.0000x reference)
import functools

import jax
import jax.numpy as jnp
from jax import lax
from jax.experimental import pallas as pl
from jax.experimental.pallas import tpu as pltpu
from jax.experimental.pallas import tpu_sc as plsc

_NC = 2
_NS = 16
_K = 80
_DW = 16
_NBUF = 5


def _make_agg(n, e, h):
    hh = h // _NC
    ept = e // _NS
    nchunk = ept // _K
    ngroup = nchunk // _NBUF
    assert ngroup * _NBUF == nchunk
    rpt = n // _NS
    mesh = plsc.VectorSubcoreMesh(core_axis_name="c", subcore_axis_name="s")

    scratch = [
        pltpu.VMEM((nchunk, _K), jnp.int32),
        pltpu.VMEM((nchunk, _K), jnp.int32),
        pltpu.VMEM((_NBUF, _K, hh), jnp.float32),
        pltpu.VMEM_SHARED((n, hh), jnp.float32),
    ] + [pltpu.SemaphoreType.DMA] * (2 * _NBUF)

    @functools.partial(
        pl.kernel, mesh=mesh,
        out_type=jax.ShapeDtypeStruct((_NC, n, hh), jnp.float32),
        scratch_types=tuple(scratch),
        compiler_params=pltpu.CompilerParams(use_tc_tiling_on_sc=False),
    )
    def agg(z_hbm, src_hbm, dst_hbm, out_hbm, sidx, didx, rows, acc_sh,
            *sems):
        semg = sems[:_NBUF]
        semsc = sems[_NBUF:]
        c = lax.axis_index("c")
        s = lax.axis_index("s")
        r0 = s * rpt
        zc = z_hbm.at[c]

        pltpu.sync_copy(src_hbm.at[s], sidx)
        pltpu.sync_copy(dst_hbm.at[s], didx)
        for b in range(_NBUF):
            pltpu.async_copy(zc.at[sidx.at[b]], rows.at[b], semg[b])
        pltpu.sync_copy(zc.at[pl.ds(r0, rpt)], acc_sh.at[pl.ds(r0, rpt)])
        plsc.subcore_barrier()

        def group(g, carry):
            scats = []
            for b in range(_NBUF):
                ci = g * _NBUF + b
                pltpu.make_async_copy(
                    zc.at[sidx.at[ci]], rows.at[b], semg[b]).wait()
                scats.append(pltpu.async_copy(
                    rows.at[b], acc_sh.at[didx.at[ci]], semsc[b], add=True))
            for b in range(_NBUF):
                scats[b].wait()
                cin = (g + 1) * _NBUF + b

                @pl.when(cin < nchunk)
                def _():
                    pltpu.async_copy(zc.at[sidx.at[cin]], rows.at[b],
                                     semg[b])
            return carry

        lax.fori_loop(0, ngroup, group, 0)
        plsc.subcore_barrier()
        pltpu.sync_copy(acc_sh.at[pl.ds(r0, rpt)],
                        out_hbm.at[c, pl.ds(r0, rpt)])

    return agg


def _make_deg(n, e):
    nw = _NC * _NS
    epw = e // nw
    nchunk = epw // _K
    rpt = n // _NS
    mesh = plsc.VectorSubcoreMesh(core_axis_name="c", subcore_axis_name="s")

    scratch = [
        pltpu.VMEM((nchunk, _K), jnp.int32),
        pltpu.VMEM((_K, _DW), jnp.float32),
        pltpu.VMEM_SHARED((n, _DW), jnp.float32),
        pltpu.SemaphoreType.DMA,
    ]

    @functools.partial(
        pl.kernel, mesh=mesh,
        out_type=jax.ShapeDtypeStruct((_NC, n, _DW), jnp.float32),
        scratch_types=tuple(scratch),
        compiler_params=pltpu.CompilerParams(use_tc_tiling_on_sc=False),
    )
    def deg(dsts_hbm, ones_hbm, half_hbm, out_hbm, didx, ones_v, deg_sh, sem):
        c = lax.axis_index("c")
        s = lax.axis_index("s")
        wid = s * _NC + c
        r0 = s * rpt

        pltpu.sync_copy(dsts_hbm.at[wid], didx)
        pltpu.sync_copy(ones_hbm, ones_v)
        pltpu.sync_copy(half_hbm.at[pl.ds(r0, rpt)],
                        deg_sh.at[pl.ds(r0, rpt)])
        plsc.subcore_barrier()

        def fire(ci, carry):
            pltpu.async_copy(ones_v, deg_sh.at[didx.at[ci]], sem, add=True)
            return carry

        lax.fori_loop(0, nchunk, fire, 0)

        def drain(ci, carry):
            pltpu.make_async_copy(ones_v, deg_sh.at[didx.at[0]], sem).wait()
            return carry

        lax.fori_loop(0, nchunk, drain, 0)
        plsc.subcore_barrier()
        pltpu.sync_copy(deg_sh.at[pl.ds(r0, rpt)],
                        out_hbm.at[c, pl.ds(r0, rpt)])

    return deg


_BN = 1000


def _row_spec(d1, bn=_BN):
    return pl.BlockSpec((bn, d1), lambda i: (i, 0))


def _pair_spec(d1, bn=_BN):
    return pl.BlockSpec((_NC, bn, d1), lambda i: (0, i, 0))


def _full_spec(shape):
    nd = len(shape)
    return pl.BlockSpec(shape, lambda i: (0,) * nd)


def _split_pair(zn_ref, zn, h):
    hh = h // _NC
    zn_ref[0] = zn[:, :hh]
    zn_ref[1] = zn[:, hh:]


def _stage_a(n, d, h):
    def body(x_ref, inw, inb, c0w, tx_ref, z0_ref):
        tx = jnp.maximum(
            jnp.dot(x_ref[...], inw[...],
                    preferred_element_type=jnp.float32) + inb[...], 0.0)
        tx_ref[...] = tx
        z0 = jnp.dot(tx, c0w[...], preferred_element_type=jnp.float32)
        _split_pair(z0_ref, z0, h)

    return pl.pallas_call(
        body,
        grid=(n // _BN,),
        in_specs=[_row_spec(d), _full_spec((d, h)), _full_spec((1, h)),
                  _full_spec((h, h))],
        out_specs=[_row_spec(h), _pair_spec(h // _NC)],
        out_shape=[jax.ShapeDtypeStruct((n, h), jnp.float32),
                   jax.ShapeDtypeStruct((_NC, n, h // _NC), jnp.float32)],
    )


def _stage_mid(n, h, first):
    def body(p_ref, d_ref, hp_ref, tx_ref, cb, cmWa, cmWb, cmWc,
             cmb, cnWa, cnWb, h_ref, zn_ref):
        invd = 1.0 / (d_ref[0, :, 0:1] + d_ref[1, :, 0:1])
        a = (jnp.concatenate([p_ref[0], p_ref[1]], axis=1) * invd + cb[...])
        acc = jnp.dot(hp_ref[...], cmWa[...], preferred_element_type=jnp.float32)
        if not first:
            acc = acc + jnp.dot(tx_ref[...], cmWb[...],
                                preferred_element_type=jnp.float32)
        hcur = jnp.tanh(acc + jnp.dot(a, cmWc[...],
                                      preferred_element_type=jnp.float32)
                        + cmb[...])
        h_ref[...] = hcur
        zn = (jnp.dot(hcur, cnWa[...], preferred_element_type=jnp.float32)
              + jnp.dot(tx_ref[...], cnWb[...],
                        preferred_element_type=jnp.float32))
        _split_pair(zn_ref, zn, h)

    return pl.pallas_call(
        body,
        grid=(n // _BN,),
        in_specs=[_pair_spec(h // _NC), _pair_spec(_DW), _row_spec(h),
                  _row_spec(h), _full_spec((1, h)), _full_spec((h, h)),
                  _full_spec((h, h)), _full_spec((h, h)), _full_spec((1, h)),
                  _full_spec((h, h)), _full_spec((h, h))],
        out_specs=[_row_spec(h), _pair_spec(h // _NC)],
        out_shape=[jax.ShapeDtypeStruct((n, h), jnp.float32),
                   jax.ShapeDtypeStruct((_NC, n, h // _NC), jnp.float32)],
    )


def _stage_last(n, h, c):
    def body(p_ref, d_ref, hp_ref, tx_ref, cb, cmWa, cmWb, cmWc,
             cmb, clfw, clfb, h_ref, y_ref):
        invd = 1.0 / (d_ref[0, :, 0:1] + d_ref[1, :, 0:1])
        a = (jnp.concatenate([p_ref[0], p_ref[1]], axis=1) * invd + cb[...])
        hcur = jnp.tanh(
            jnp.dot(hp_ref[...], cmWa[...], preferred_element_type=jnp.float32)
            + jnp.dot(tx_ref[...], cmWb[...], preferred_element_type=jnp.float32)
            + jnp.dot(a, cmWc[...], preferred_element_type=jnp.float32)
            + cmb[...])
        h_ref[...] = hcur
        y_ref[...] = jnp.dot(hcur, clfw[...],
                             preferred_element_type=jnp.float32) + clfb[...]

    return pl.pallas_call(
        body,
        grid=(n // _BN,),
        in_specs=[_pair_spec(h // _NC), _pair_spec(_DW), _row_spec(h),
                  _row_spec(h), _full_spec((1, h)), _full_spec((h, h)),
                  _full_spec((h, h)), _full_spec((h, h)), _full_spec((1, h)),
                  _full_spec((h, c)), _full_spec((1, c))],
        out_specs=[_row_spec(h), _row_spec(c)],
        out_shape=[jax.ShapeDtypeStruct((n, h), jnp.float32),
                   jax.ShapeDtypeStruct((n, c), jnp.float32)],
    )


def kernel(x, edge_index, in_W, in_b, conv0_W, conv0_b, conv1_W, conv1_b,
           conv2_W, conv2_b, comb0_W, comb0_b, comb1_W, comb1_b,
           comb2_W, comb2_b, clf_W, clf_b):
    n, d = x.shape
    e = edge_index.shape[1]
    h = in_W.shape[1]
    c = clf_W.shape[1]
    nw = _NC * _NS
    assert e % (nw * _K) == 0 and n % _NS == 0 and n % _BN == 0
    assert (e // _NS // _K) % _NBUF == 0 and h % _NC == 0

    srcs = edge_index[0].reshape(_NS, -1, _K)
    dsts = edge_index[1].reshape(_NS, -1, _K)
    dstw = edge_index[1].reshape(nw, -1, _K)
    ones = jnp.ones((_K, _DW), jnp.float32)
    half = jnp.full((n, _DW), 0.5, jnp.float32)

    agg = _make_agg(n, e, h)
    r1 = lambda b: b.reshape(1, -1)

    deg = _make_deg(n, e)(dstw, ones, half)
    tx, z0 = _stage_a(n, d, h)(x, in_W, r1(in_b), conv0_W)
    p0 = agg(z0, srcs, dsts)
    h0, z1 = _stage_mid(n, h, first=True)(
        p0, deg, tx, tx, r1(conv0_b),
        comb0_W[:h], comb0_W[:h], comb0_W[h:], r1(comb0_b),
        conv1_W[:h], conv1_W[h:])
    p1 = agg(z1, srcs, dsts)
    h1, z2 = _stage_mid(n, h, first=False)(
        p1, deg, h0, tx, r1(conv1_b),
        comb1_W[:h], comb1_W[h:2 * h], comb1_W[2 * h:], r1(comb1_b),
        conv2_W[:h], conv2_W[h:])
    p2 = agg(z2, srcs, dsts)
    h2, y = _stage_last(n, h, c)(
        p2, deg, h1, tx, r1(conv2_b),
        comb2_W[:h], comb2_W[h:2 * h], comb2_W[2 * h:], r1(comb2_b),
        clf_W, r1(clf_b))
    return (h0, h1, h2, y)


# speedup vs baseline: 1.0814x; 1.0814x over previous
import functools

import jax
import jax.numpy as jnp
from jax import lax
from jax.experimental import pallas as pl
from jax.experimental.pallas import tpu as pltpu
from jax.experimental.pallas import tpu_sc as plsc

_NC = 2
_NS = 16
_K = 125
_DW = 16
_NBUF = 5


def _make_agg(n, e, h):
    hh = h // _NC
    ept = e // _NS
    nchunk = ept // _K
    ngroup = nchunk // _NBUF
    assert ngroup * _NBUF == nchunk
    rpt = n // _NS
    mesh = plsc.VectorSubcoreMesh(core_axis_name="c", subcore_axis_name="s")

    scratch = [
        pltpu.VMEM((2, nchunk, _K), jnp.int32),
        pltpu.VMEM((_NBUF, _K, hh), jnp.float32),
        pltpu.VMEM_SHARED((n, hh), jnp.float32),
    ] + [pltpu.SemaphoreType.DMA] * (2 * _NBUF)

    @functools.partial(
        pl.kernel, mesh=mesh,
        out_type=jax.ShapeDtypeStruct((_NC, n, hh), jnp.float32),
        scratch_types=tuple(scratch),
        compiler_params=pltpu.CompilerParams(use_tc_tiling_on_sc=False),
    )
    def agg(z_hbm, ei_hbm, out_hbm, sd, rows, acc_sh, *sems):
        semg = sems[:_NBUF]
        semsc = sems[_NBUF:]
        c = lax.axis_index("c")
        s = lax.axis_index("s")
        r0 = s * rpt
        zc = z_hbm.at[c]

        pltpu.sync_copy(ei_hbm.at[pl.ds(0, 2), s], sd)
        for b in range(_NBUF):
            pltpu.async_copy(zc.at[sd.at[0, b]], rows.at[b], semg[b])
        pltpu.sync_copy(zc.at[pl.ds(r0, rpt)], acc_sh.at[pl.ds(r0, rpt)])
        plsc.subcore_barrier()

        def group(g, carry):
            scats = []
            for b in range(_NBUF):
                ci = g * _NBUF + b
                pltpu.make_async_copy(
                    zc.at[sd.at[0, ci]], rows.at[b], semg[b]).wait()
                scats.append(pltpu.async_copy(
                    rows.at[b], acc_sh.at[sd.at[1, ci]], semsc[b], add=True))
            for b in range(_NBUF):
                scats[b].wait()
                cin = (g + 1) * _NBUF + b

                @pl.when(cin < nchunk)
                def _():
                    pltpu.async_copy(zc.at[sd.at[0, cin]], rows.at[b],
                                     semg[b])
            return carry

        lax.fori_loop(0, ngroup, group, 0)
        plsc.subcore_barrier()
        pltpu.sync_copy(acc_sh.at[pl.ds(r0, rpt)],
                        out_hbm.at[c, pl.ds(r0, rpt)])

    return agg


def _make_deg(n, e):
    ept = e // _NS
    nchunk = ept // _K
    nch = nchunk // _NC
    rpt = n // _NS
    mesh = plsc.VectorSubcoreMesh(core_axis_name="c", subcore_axis_name="s")

    scratch = [
        pltpu.VMEM((nch, _K), jnp.int32),
        pltpu.VMEM((_K, _DW), jnp.float32),
        pltpu.VMEM_SHARED((n, _DW), jnp.float32),
        pltpu.SemaphoreType.DMA,
    ]

    @functools.partial(
        pl.kernel, mesh=mesh,
        out_type=jax.ShapeDtypeStruct((_NC, n, _DW), jnp.float32),
        scratch_types=tuple(scratch),
        compiler_params=pltpu.CompilerParams(use_tc_tiling_on_sc=False),
    )
    def deg(ei_hbm, ones_hbm, half_hbm, out_hbm, didx, ones_v, deg_sh, sem):
        c = lax.axis_index("c")
        s = lax.axis_index("s")
        r0 = s * rpt

        pltpu.sync_copy(ei_hbm.at[1, s, pl.ds(c * nch, nch)], didx)
        pltpu.sync_copy(ones_hbm, ones_v)
        pltpu.sync_copy(half_hbm.at[pl.ds(r0, rpt)],
                        deg_sh.at[pl.ds(r0, rpt)])
        plsc.subcore_barrier()

        def fire(ci, carry):
            pltpu.async_copy(ones_v, deg_sh.at[didx.at[ci]], sem, add=True)
            return carry

        lax.fori_loop(0, nch, fire, 0)

        def drain(ci, carry):
            pltpu.make_async_copy(ones_v, deg_sh.at[didx.at[0]], sem).wait()
            return carry

        lax.fori_loop(0, nch, drain, 0)
        plsc.subcore_barrier()
        pltpu.sync_copy(deg_sh.at[pl.ds(r0, rpt)],
                        out_hbm.at[c, pl.ds(r0, rpt)])

    return deg


_BN = 2000


def _row_spec(d1, bn=_BN):
    return pl.BlockSpec((bn, d1), lambda i: (i, 0))


def _pair_spec(d1, bn=_BN):
    return pl.BlockSpec((_NC, bn, d1), lambda i: (0, i, 0))


def _full_spec(shape):
    nd = len(shape)
    return pl.BlockSpec(shape, lambda i: (0,) * nd)


def _wblk(h, j):
    return pl.BlockSpec((h, h), lambda i, j=j: (j, 0))


def _split_pair(zn_ref, zn, h):
    hh = h // _NC
    zn_ref[0] = zn[:, :hh]
    zn_ref[1] = zn[:, hh:]


def _stage_a(n, d, h):
    def body(x_ref, inw, inb, c0w, tx_ref, z0_ref):
        tx = jnp.maximum(
            jnp.dot(x_ref[...], inw[...],
                    preferred_element_type=jnp.float32) + inb[...], 0.0)
        tx_ref[...] = tx
        z0 = jnp.dot(tx, c0w[...], preferred_element_type=jnp.float32)
        _split_pair(z0_ref, z0, h)

    return pl.pallas_call(
        body,
        grid=(n // _BN,),
        in_specs=[_row_spec(d), _full_spec((d, h)), _full_spec((1, h)),
                  _full_spec((h, h))],
        out_specs=[_row_spec(h), _pair_spec(h // _NC)],
        out_shape=[jax.ShapeDtypeStruct((n, h), jnp.float32),
                   jax.ShapeDtypeStruct((_NC, n, h // _NC), jnp.float32)],
    )


def _stage_mid(n, h, first):
    def body(p_ref, d_ref, hp_ref, tx_ref, cb, cmWa, cmWb, cmWc,
             cmb, cnWa, cnWb, h_ref, zn_ref):
        invd = 1.0 / (d_ref[0, :, 0:1] + d_ref[1, :, 0:1])
        a = (jnp.concatenate([p_ref[0], p_ref[1]], axis=1) * invd + cb[...])
        acc = jnp.dot(hp_ref[...], cmWa[...], preferred_element_type=jnp.float32)
        if not first:
            acc = acc + jnp.dot(tx_ref[...], cmWb[...],
                                preferred_element_type=jnp.float32)
        hcur = jnp.tanh(acc + jnp.dot(a, cmWc[...],
                                      preferred_element_type=jnp.float32)
                        + cmb[...])
        h_ref[...] = hcur
        zn = (jnp.dot(hcur, cnWa[...], preferred_element_type=jnp.float32)
              + jnp.dot(tx_ref[...], cnWb[...],
                        preferred_element_type=jnp.float32))
        _split_pair(zn_ref, zn, h)

    nw_c = 2 if first else 3
    return pl.pallas_call(
        body,
        grid=(n // _BN,),
        in_specs=[_pair_spec(h // _NC), _pair_spec(_DW), _row_spec(h),
                  _row_spec(h), _full_spec((1, h)), _wblk(h, 0),
                  _wblk(h, 1 if not first else 0), _wblk(h, nw_c - 1),
                  _full_spec((1, h)), _wblk(h, 0), _wblk(h, 1)],
        out_specs=[_row_spec(h), _pair_spec(h // _NC)],
        out_shape=[jax.ShapeDtypeStruct((n, h), jnp.float32),
                   jax.ShapeDtypeStruct((_NC, n, h // _NC), jnp.float32)],
    )


def _stage_last(n, h, c):
    def body(p_ref, d_ref, hp_ref, tx_ref, cb, cmWa, cmWb, cmWc,
             cmb, clfw, clfb, h_ref, y_ref):
        invd = 1.0 / (d_ref[0, :, 0:1] + d_ref[1, :, 0:1])
        a = (jnp.concatenate([p_ref[0], p_ref[1]], axis=1) * invd + cb[...])
        hcur = jnp.tanh(
            jnp.dot(hp_ref[...], cmWa[...], preferred_element_type=jnp.float32)
            + jnp.dot(tx_ref[...], cmWb[...], preferred_element_type=jnp.float32)
            + jnp.dot(a, cmWc[...], preferred_element_type=jnp.float32)
            + cmb[...])
        h_ref[...] = hcur
        y_ref[...] = jnp.dot(hcur, clfw[...],
                             preferred_element_type=jnp.float32) + clfb[...]

    return pl.pallas_call(
        body,
        grid=(n // _BN,),
        in_specs=[_pair_spec(h // _NC), _pair_spec(_DW), _row_spec(h),
                  _row_spec(h), _full_spec((1, h)), _wblk(h, 0),
                  _wblk(h, 1), _wblk(h, 2), _full_spec((1, h)),
                  _full_spec((h, c)), _full_spec((1, c))],
        out_specs=[_row_spec(h), _row_spec(c)],
        out_shape=[jax.ShapeDtypeStruct((n, h), jnp.float32),
                   jax.ShapeDtypeStruct((n, c), jnp.float32)],
    )


def kernel(x, edge_index, in_W, in_b, conv0_W, conv0_b, conv1_W, conv1_b,
           conv2_W, conv2_b, comb0_W, comb0_b, comb1_W, comb1_b,
           comb2_W, comb2_b, clf_W, clf_b):
    n, d = x.shape
    e = edge_index.shape[1]
    h = in_W.shape[1]
    c = clf_W.shape[1]
    assert e % (_NS * _K) == 0 and n % _NS == 0 and n % _BN == 0
    assert (e // _NS // _K) % (_NBUF * _NC) == 0 and h % _NC == 0

    ei = edge_index.reshape(2, _NS, -1, _K)
    ones = jnp.ones((_K, _DW), jnp.float32)
    half = jnp.full((n, _DW), 0.5, jnp.float32)

    agg = _make_agg(n, e, h)
    r1 = lambda b: b.reshape(1, -1)

    deg = _make_deg(n, e)(ei, ones, half)
    tx, z0 = _stage_a(n, d, h)(x, in_W, r1(in_b), conv0_W)
    p0 = agg(z0, ei)
    h0, z1 = _stage_mid(n, h, first=True)(
        p0, deg, tx, tx, r1(conv0_b),
        comb0_W, comb0_W, comb0_W, r1(comb0_b),
        conv1_W, conv1_W)
    p1 = agg(z1, ei)
    h1, z2 = _stage_mid(n, h, first=False)(
        p1, deg, h0, tx, r1(conv1_b),
        comb1_W, comb1_W, comb1_W, r1(comb1_b),
        conv2_W, conv2_W)
    p2 = agg(z2, ei)
    h2, y = _stage_last(n, h, c)(
        p2, deg, h1, tx, r1(conv2_b),
        comb2_W, comb2_W, comb2_W, r1(comb2_b),
        clf_W, r1(clf_b))
    return (h0, h1, h2, y)


# speedup vs baseline: 1.0827x; 1.0012x over previous
import functools

import jax
import jax.numpy as jnp
from jax import lax
from jax.experimental import pallas as pl
from jax.experimental.pallas import tpu as pltpu
from jax.experimental.pallas import tpu_sc as plsc

_NC = 2
_NS = 16
_K = 125
_DW = 16
_NBUF = 5


def _make_agg(n, e, h):
    hh = h // _NC
    ept = e // _NS
    nchunk = ept // _K
    ngroup = nchunk // _NBUF
    assert ngroup * _NBUF == nchunk
    rpt = n // _NS
    mesh = plsc.VectorSubcoreMesh(core_axis_name="c", subcore_axis_name="s")

    scratch = [
        pltpu.VMEM((2, nchunk, _K), jnp.int32),
        pltpu.VMEM((_NBUF, _K, hh), jnp.float32),
        pltpu.VMEM_SHARED((n, hh), jnp.float32),
    ] + [pltpu.SemaphoreType.DMA] * (2 * _NBUF)

    @functools.partial(
        pl.kernel, mesh=mesh,
        out_type=jax.ShapeDtypeStruct((_NC, n, hh), jnp.float32),
        scratch_types=tuple(scratch),
        compiler_params=pltpu.CompilerParams(use_tc_tiling_on_sc=False),
    )
    def agg(z_hbm, ei_hbm, out_hbm, sd, rows, acc_sh, *sems):
        semg = sems[:_NBUF]
        semsc = sems[_NBUF:]
        c = lax.axis_index("c")
        s = lax.axis_index("s")
        r0 = s * rpt
        zc = z_hbm.at[c]

        pltpu.sync_copy(ei_hbm.at[pl.ds(0, 2), s], sd)
        for b in range(_NBUF):
            pltpu.async_copy(zc.at[sd.at[0, b]], rows.at[b], semg[b])
        pltpu.sync_copy(zc.at[pl.ds(r0, rpt)], acc_sh.at[pl.ds(r0, rpt)])
        plsc.subcore_barrier()

        def group(g, carry):
            scats = []
            for b in range(_NBUF):
                ci = g * _NBUF + b
                pltpu.make_async_copy(
                    zc.at[sd.at[0, ci]], rows.at[b], semg[b]).wait()
                scats.append(pltpu.async_copy(
                    rows.at[b], acc_sh.at[sd.at[1, ci]], semsc[b], add=True))
            for b in range(_NBUF):
                scats[b].wait()
                cin = (g + 1) * _NBUF + b

                @pl.when(cin < nchunk)
                def _():
                    pltpu.async_copy(zc.at[sd.at[0, cin]], rows.at[b],
                                     semg[b])
            return carry

        lax.fori_loop(0, ngroup, group, 0)
        plsc.subcore_barrier()
        pltpu.sync_copy(acc_sh.at[pl.ds(r0, rpt)],
                        out_hbm.at[c, pl.ds(r0, rpt)])

    return agg


def _make_deg(n, e):
    ept = e // _NS
    nchunk = ept // _K
    nch = nchunk // _NC
    rpt = n // _NS
    mesh = plsc.VectorSubcoreMesh(core_axis_name="c", subcore_axis_name="s")

    scratch = [
        pltpu.VMEM((nch, _K), jnp.int32),
        pltpu.VMEM((_K, _DW), jnp.float32),
        pltpu.VMEM_SHARED((n, _DW), jnp.float32),
        pltpu.SemaphoreType.DMA,
    ]

    @functools.partial(
        pl.kernel, mesh=mesh,
        out_type=jax.ShapeDtypeStruct((_NC, n, _DW), jnp.float32),
        scratch_types=tuple(scratch),
        compiler_params=pltpu.CompilerParams(use_tc_tiling_on_sc=False),
    )
    def deg(ei_hbm, ones_hbm, half_hbm, out_hbm, didx, ones_v, deg_sh, sem):
        c = lax.axis_index("c")
        s = lax.axis_index("s")
        r0 = s * rpt

        pltpu.sync_copy(ei_hbm.at[1, s, pl.ds(c * nch, nch)], didx)
        pltpu.sync_copy(ones_hbm, ones_v)
        pltpu.sync_copy(half_hbm.at[pl.ds(r0, rpt)],
                        deg_sh.at[pl.ds(r0, rpt)])
        plsc.subcore_barrier()

        def fire(ci, carry):
            pltpu.async_copy(ones_v, deg_sh.at[didx.at[ci]], sem, add=True)
            return carry

        lax.fori_loop(0, nch, fire, 0)

        def drain(ci, carry):
            pltpu.make_async_copy(ones_v, deg_sh.at[didx.at[0]], sem).wait()
            return carry

        lax.fori_loop(0, nch, drain, 0)
        plsc.subcore_barrier()
        pltpu.sync_copy(deg_sh.at[pl.ds(r0, rpt)],
                        out_hbm.at[c, pl.ds(r0, rpt)])

    return deg


_BN = 2000


def _row_spec(d1, bn=_BN):
    return pl.BlockSpec((bn, d1), lambda i: (i, 0))


def _pair_spec(d1, bn=_BN):
    return pl.BlockSpec((_NC, bn, d1), lambda i: (0, i, 0))


def _full_spec(shape):
    nd = len(shape)
    return pl.BlockSpec(shape, lambda i: (0,) * nd)


def _wblk(h, j):
    return pl.BlockSpec((h, h), lambda i, j=j: (j, 0))


def _split_pair(zn_ref, zn, h):
    hh = h // _NC
    zn_ref[0] = zn[:, :hh]
    zn_ref[1] = zn[:, hh:]


def _stage_a(n, d, h):
    def body(x_ref, inw, inb, c0w, tx_ref, z0_ref):
        tx = jnp.maximum(
            jnp.dot(x_ref[...], inw[...],
                    preferred_element_type=jnp.float32) + inb[...], 0.0)
        tx_ref[...] = tx
        z0 = jnp.dot(tx, c0w[...], preferred_element_type=jnp.float32)
        _split_pair(z0_ref, z0, h)

    return pl.pallas_call(
        body,
        grid=(n // _BN,),
        in_specs=[_row_spec(d), _full_spec((d, h)), _full_spec((h,)),
                  _full_spec((h, h))],
        out_specs=[_row_spec(h), _pair_spec(h // _NC)],
        out_shape=[jax.ShapeDtypeStruct((n, h), jnp.float32),
                   jax.ShapeDtypeStruct((_NC, n, h // _NC), jnp.float32)],
    )


def _stage_mid(n, h, first):
    def body(p_ref, d_ref, hp_ref, tx_ref, cb, cmWa, cmWb, cmWc,
             cmb, cnWa, cnWb, h_ref, zn_ref):
        invd = 1.0 / (d_ref[0, :, 0:1] + d_ref[1, :, 0:1])
        a = (jnp.concatenate([p_ref[0], p_ref[1]], axis=1) * invd + cb[...])
        acc = jnp.dot(hp_ref[...], cmWa[...], preferred_element_type=jnp.float32)
        if not first:
            acc = acc + jnp.dot(tx_ref[...], cmWb[...],
                                preferred_element_type=jnp.float32)
        hcur = jnp.tanh(acc + jnp.dot(a, cmWc[...],
                                      preferred_element_type=jnp.float32)
                        + cmb[...])
        h_ref[...] = hcur
        zn = (jnp.dot(hcur, cnWa[...], preferred_element_type=jnp.float32)
              + jnp.dot(tx_ref[...], cnWb[...],
                        preferred_element_type=jnp.float32))
        _split_pair(zn_ref, zn, h)

    nw_c = 2 if first else 3
    return pl.pallas_call(
        body,
        grid=(n // _BN,),
        in_specs=[_pair_spec(h // _NC), _pair_spec(_DW), _row_spec(h),
                  _row_spec(h), _full_spec((h,)), _wblk(h, 0),
                  _wblk(h, 1 if not first else 0), _wblk(h, nw_c - 1),
                  _full_spec((h,)), _wblk(h, 0), _wblk(h, 1)],
        out_specs=[_row_spec(h), _pair_spec(h // _NC)],
        out_shape=[jax.ShapeDtypeStruct((n, h), jnp.float32),
                   jax.ShapeDtypeStruct((_NC, n, h // _NC), jnp.float32)],
    )


def _stage_last(n, h, c):
    def body(p_ref, d_ref, hp_ref, tx_ref, cb, cmWa, cmWb, cmWc,
             cmb, clfw, clfb, h_ref, y_ref):
        invd = 1.0 / (d_ref[0, :, 0:1] + d_ref[1, :, 0:1])
        a = (jnp.concatenate([p_ref[0], p_ref[1]], axis=1) * invd + cb[...])
        hcur = jnp.tanh(
            jnp.dot(hp_ref[...], cmWa[...], preferred_element_type=jnp.float32)
            + jnp.dot(tx_ref[...], cmWb[...], preferred_element_type=jnp.float32)
            + jnp.dot(a, cmWc[...], preferred_element_type=jnp.float32)
            + cmb[...])
        h_ref[...] = hcur
        y_ref[...] = jnp.dot(hcur, clfw[...],
                             preferred_element_type=jnp.float32) + clfb[...]

    return pl.pallas_call(
        body,
        grid=(n // _BN,),
        in_specs=[_pair_spec(h // _NC), _pair_spec(_DW), _row_spec(h),
                  _row_spec(h), _full_spec((h,)), _wblk(h, 0),
                  _wblk(h, 1), _wblk(h, 2), _full_spec((h,)),
                  _full_spec((h, c)), _full_spec((c,))],
        out_specs=[_row_spec(h), _row_spec(c)],
        out_shape=[jax.ShapeDtypeStruct((n, h), jnp.float32),
                   jax.ShapeDtypeStruct((n, c), jnp.float32)],
    )


def kernel(x, edge_index, in_W, in_b, conv0_W, conv0_b, conv1_W, conv1_b,
           conv2_W, conv2_b, comb0_W, comb0_b, comb1_W, comb1_b,
           comb2_W, comb2_b, clf_W, clf_b):
    n, d = x.shape
    e = edge_index.shape[1]
    h = in_W.shape[1]
    c = clf_W.shape[1]
    assert e % (_NS * _K) == 0 and n % _NS == 0 and n % _BN == 0
    assert (e // _NS // _K) % (_NBUF * _NC) == 0 and h % _NC == 0

    ei = edge_index.reshape(2, _NS, -1, _K)
    ones = jnp.ones((_K, _DW), jnp.float32)
    half = jnp.full((n, _DW), 0.5, jnp.float32)

    agg = _make_agg(n, e, h)

    deg = _make_deg(n, e)(ei, ones, half)
    tx, z0 = _stage_a(n, d, h)(x, in_W, in_b, conv0_W)
    p0 = agg(z0, ei)
    h0, z1 = _stage_mid(n, h, first=True)(
        p0, deg, tx, tx, conv0_b,
        comb0_W, comb0_W, comb0_W, comb0_b,
        conv1_W, conv1_W)
    p1 = agg(z1, ei)
    h1, z2 = _stage_mid(n, h, first=False)(
        p1, deg, h0, tx, conv1_b,
        comb1_W, comb1_W, comb1_W, comb1_b,
        conv2_W, conv2_W)
    p2 = agg(z2, ei)
    h2, y = _stage_last(n, h, c)(
        p2, deg, h1, tx, conv2_b,
        comb2_W, comb2_W, comb2_W, comb2_b,
        clf_W, clf_b)
    return (h0, h1, h2, y)


# speedup vs baseline: 1.1585x; 1.0700x over previous
import functools

import jax
import jax.numpy as jnp
from jax import lax
from jax.experimental import pallas as pl
from jax.experimental.pallas import tpu as pltpu
from jax.experimental.pallas import tpu_sc as plsc

_NC = 2
_NS = 16
_K = 125
_DW = 16
_NBUF = 5


def _make_agg(n, e, h):
    hh = h // _NC
    ept = e // _NS
    nchunk = ept // _K
    ngroup = nchunk // _NBUF
    assert ngroup * _NBUF == nchunk
    rpt = n // _NS
    mesh = plsc.VectorSubcoreMesh(core_axis_name="c", subcore_axis_name="s")

    scratch = [
        pltpu.VMEM((2, nchunk, _K), jnp.int32),
        pltpu.VMEM((_NBUF, _K, hh), jnp.float32),
        pltpu.VMEM_SHARED((n, hh), jnp.float32),
    ] + [pltpu.SemaphoreType.DMA] * (2 * _NBUF)

    @functools.partial(
        pl.kernel, mesh=mesh,
        out_type=jax.ShapeDtypeStruct((n, h), jnp.float32),
        scratch_types=tuple(scratch),
        compiler_params=pltpu.CompilerParams(use_tc_tiling_on_sc=False),
    )
    def agg(z_hbm, ei_hbm, out_hbm, sd, rows, acc_sh, *sems):
        semg = sems[:_NBUF]
        semsc = sems[_NBUF:]
        c = lax.axis_index("c")
        s = lax.axis_index("s")
        r0 = s * rpt
        zc = z_hbm.at[c]

        pltpu.sync_copy(ei_hbm.at[pl.ds(0, 2), s], sd)
        for b in range(_NBUF):
            pltpu.async_copy(zc.at[sd.at[0, b]], rows.at[b], semg[b])
        pltpu.sync_copy(zc.at[pl.ds(r0, rpt)], acc_sh.at[pl.ds(r0, rpt)])
        plsc.subcore_barrier()

        def group(g, carry):
            scats = []
            for b in range(_NBUF):
                ci = g * _NBUF + b
                pltpu.make_async_copy(
                    zc.at[sd.at[0, ci]], rows.at[b], semg[b]).wait()
                scats.append(pltpu.async_copy(
                    rows.at[b], acc_sh.at[sd.at[1, ci]], semsc[b], add=True))
            for b in range(_NBUF):
                scats[b].wait()
                cin = (g + 1) * _NBUF + b

                @pl.when(cin < nchunk)
                def _():
                    pltpu.async_copy(zc.at[sd.at[0, cin]], rows.at[b],
                                     semg[b])
            return carry

        lax.fori_loop(0, ngroup, group, 0)
        plsc.subcore_barrier()
        pltpu.sync_copy(acc_sh.at[pl.ds(r0, rpt)],
                        out_hbm.at[pl.ds(r0, rpt), pl.ds(c * hh, hh)])

    return agg


def _make_deg(n, e):
    ept = e // _NS
    nchunk = ept // _K
    nch = nchunk // _NC
    rpt = n // _NS
    mesh = plsc.VectorSubcoreMesh(core_axis_name="c", subcore_axis_name="s")

    scratch = [
        pltpu.VMEM((nch, _K), jnp.int32),
        pltpu.VMEM((_K, _DW), jnp.float32),
        pltpu.VMEM_SHARED((n, _DW), jnp.float32),
        pltpu.SemaphoreType.DMA,
    ]

    @functools.partial(
        pl.kernel, mesh=mesh,
        out_type=jax.ShapeDtypeStruct((_NC, n, _DW), jnp.float32),
        scratch_types=tuple(scratch),
        compiler_params=pltpu.CompilerParams(use_tc_tiling_on_sc=False),
    )
    def deg(ei_hbm, ones_hbm, half_hbm, out_hbm, didx, ones_v, deg_sh, sem):
        c = lax.axis_index("c")
        s = lax.axis_index("s")
        r0 = s * rpt

        pltpu.sync_copy(ei_hbm.at[1, s, pl.ds(c * nch, nch)], didx)
        pltpu.sync_copy(ones_hbm, ones_v)
        pltpu.sync_copy(half_hbm.at[pl.ds(r0, rpt)],
                        deg_sh.at[pl.ds(r0, rpt)])
        plsc.subcore_barrier()

        def fire(ci, carry):
            pltpu.async_copy(ones_v, deg_sh.at[didx.at[ci]], sem, add=True)
            return carry

        lax.fori_loop(0, nch, fire, 0)

        def drain(ci, carry):
            pltpu.make_async_copy(ones_v, deg_sh.at[didx.at[0]], sem).wait()
            return carry

        lax.fori_loop(0, nch, drain, 0)
        plsc.subcore_barrier()
        pltpu.sync_copy(deg_sh.at[pl.ds(r0, rpt)],
                        out_hbm.at[c, pl.ds(r0, rpt)])

    return deg


_BN = 2000


def _row_spec(d1, bn=_BN):
    return pl.BlockSpec((bn, d1), lambda i: (i, 0))


def _pair_spec(d1, bn=_BN):
    return pl.BlockSpec((_NC, bn, d1), lambda i: (0, i, 0))


def _full_spec(shape):
    nd = len(shape)
    return pl.BlockSpec(shape, lambda i: (0,) * nd)


def _wblk(h, j):
    return pl.BlockSpec((h, h), lambda i, j=j: (j, 0))


def _split_pair(zn_ref, zn, h):
    hh = h // _NC
    zn_ref[0] = zn[:, :hh]
    zn_ref[1] = zn[:, hh:]


def _stage_a(n, d, h):
    def body(x_ref, inw, inb, c0w, tx_ref, z0_ref):
        tx = jnp.maximum(
            jnp.dot(x_ref[...], inw[...],
                    preferred_element_type=jnp.float32) + inb[...], 0.0)
        tx_ref[...] = tx
        z0 = jnp.dot(tx, c0w[...], preferred_element_type=jnp.float32)
        _split_pair(z0_ref, z0, h)

    return pl.pallas_call(
        body,
        grid=(n // _BN,),
        in_specs=[_row_spec(d), _full_spec((d, h)), _full_spec((h,)),
                  _full_spec((h, h))],
        out_specs=[_row_spec(h), _pair_spec(h // _NC)],
        out_shape=[jax.ShapeDtypeStruct((n, h), jnp.float32),
                   jax.ShapeDtypeStruct((_NC, n, h // _NC), jnp.float32)],
    )


def _stage_mid(n, h, first):
    def body(p_ref, d_ref, hp_ref, tx_ref, cb, cmWa, cmWb, cmWc,
             cmb, cnWa, cnWb, h_ref, zn_ref):
        invd = 1.0 / (d_ref[0, :, 0:1] + d_ref[1, :, 0:1])
        a = p_ref[...] * invd + cb[...]
        acc = jnp.dot(hp_ref[...], cmWa[...], preferred_element_type=jnp.float32)
        if not first:
            acc = acc + jnp.dot(tx_ref[...], cmWb[...],
                                preferred_element_type=jnp.float32)
        hcur = jnp.tanh(acc + jnp.dot(a, cmWc[...],
                                      preferred_element_type=jnp.float32)
                        + cmb[...])
        h_ref[...] = hcur
        zn = (jnp.dot(hcur, cnWa[...], preferred_element_type=jnp.float32)
              + jnp.dot(tx_ref[...], cnWb[...],
                        preferred_element_type=jnp.float32))
        _split_pair(zn_ref, zn, h)

    nw_c = 2 if first else 3
    return pl.pallas_call(
        body,
        grid=(n // _BN,),
        in_specs=[_row_spec(h), _pair_spec(_DW), _row_spec(h),
                  _row_spec(h), _full_spec((h,)), _wblk(h, 0),
                  _wblk(h, 1 if not first else 0), _wblk(h, nw_c - 1),
                  _full_spec((h,)), _wblk(h, 0), _wblk(h, 1)],
        out_specs=[_row_spec(h), _pair_spec(h // _NC)],
        out_shape=[jax.ShapeDtypeStruct((n, h), jnp.float32),
                   jax.ShapeDtypeStruct((_NC, n, h // _NC), jnp.float32)],
    )


def _stage_last(n, h, c):
    def body(p_ref, d_ref, hp_ref, tx_ref, cb, cmWa, cmWb, cmWc,
             cmb, clfw, clfb, h_ref, y_ref):
        invd = 1.0 / (d_ref[0, :, 0:1] + d_ref[1, :, 0:1])
        a = p_ref[...] * invd + cb[...]
        hcur = jnp.tanh(
            jnp.dot(hp_ref[...], cmWa[...], preferred_element_type=jnp.float32)
            + jnp.dot(tx_ref[...], cmWb[...], preferred_element_type=jnp.float32)
            + jnp.dot(a, cmWc[...], preferred_element_type=jnp.float32)
            + cmb[...])
        h_ref[...] = hcur
        y_ref[...] = jnp.dot(hcur, clfw[...],
                             preferred_element_type=jnp.float32) + clfb[...]

    return pl.pallas_call(
        body,
        grid=(n // _BN,),
        in_specs=[_row_spec(h), _pair_spec(_DW), _row_spec(h),
                  _row_spec(h), _full_spec((h,)), _wblk(h, 0),
                  _wblk(h, 1), _wblk(h, 2), _full_spec((h,)),
                  _full_spec((h, c)), _full_spec((c,))],
        out_specs=[_row_spec(h), _row_spec(c)],
        out_shape=[jax.ShapeDtypeStruct((n, h), jnp.float32),
                   jax.ShapeDtypeStruct((n, c), jnp.float32)],
    )


def kernel(x, edge_index, in_W, in_b, conv0_W, conv0_b, conv1_W, conv1_b,
           conv2_W, conv2_b, comb0_W, comb0_b, comb1_W, comb1_b,
           comb2_W, comb2_b, clf_W, clf_b):
    n, d = x.shape
    e = edge_index.shape[1]
    h = in_W.shape[1]
    c = clf_W.shape[1]
    assert e % (_NS * _K) == 0 and n % _NS == 0 and n % _BN == 0
    assert (e // _NS // _K) % (_NBUF * _NC) == 0 and h % _NC == 0

    ei = edge_index.reshape(2, _NS, -1, _K)
    ones = jnp.ones((_K, _DW), jnp.float32)
    half = jnp.full((n, _DW), 0.5, jnp.float32)

    agg = _make_agg(n, e, h)

    deg = _make_deg(n, e)(ei, ones, half)
    tx, z0 = _stage_a(n, d, h)(x, in_W, in_b, conv0_W)
    p0 = agg(z0, ei)
    h0, z1 = _stage_mid(n, h, first=True)(
        p0, deg, tx, tx, conv0_b,
        comb0_W, comb0_W, comb0_W, comb0_b,
        conv1_W, conv1_W)
    p1 = agg(z1, ei)
    h1, z2 = _stage_mid(n, h, first=False)(
        p1, deg, h0, tx, conv1_b,
        comb1_W, comb1_W, comb1_W, comb1_b,
        conv2_W, conv2_W)
    p2 = agg(z2, ei)
    h2, y = _stage_last(n, h, c)(
        p2, deg, h1, tx, conv2_b,
        comb2_W, comb2_W, comb2_W, comb2_b,
        clf_W, clf_b)
    return (h0, h1, h2, y)
